# Initial kernel scaffold; baseline (speedup 1.0000x reference)
import jax
import jax.numpy as jnp
from jax.experimental import pallas as pl


def kernel(token_ids, embeddings):
    raise NotImplementedError("write your pallas kernel here")


# speedup vs baseline: 1.1023x; 1.1023x over previous
import functools

import jax
import jax.numpy as jnp
from jax import lax
from jax.experimental import pallas as pl
from jax.experimental.pallas import tpu as pltpu
from jax.experimental.pallas import tpu_sc as plsc

NUM_TOKENS = 16384 * 50
EMBED_DIM = 32
NUM_CORES = 2
NUM_SUBCORES = 16
NUM_WORKERS = NUM_CORES * NUM_SUBCORES
PER_WORKER = NUM_TOKENS // NUM_WORKERS
CHUNK = 1024
NUM_CHUNKS = PER_WORKER // CHUNK

_mesh = plsc.VectorSubcoreMesh(core_axis_name="c", subcore_axis_name="s")


@functools.partial(
    pl.kernel,
    mesh=_mesh,
    compiler_params=pltpu.CompilerParams(use_tc_tiling_on_sc=False),
    out_type=jax.ShapeDtypeStruct((NUM_TOKENS, EMBED_DIM), jnp.float32),
    scratch_types=[
        pltpu.VMEM((PER_WORKER,), jnp.int32),
        pltpu.VMEM((CHUNK, EMBED_DIM), jnp.float32),
        pltpu.SemaphoreType.DMA,
    ],
)
def _gather_kernel(ids_hbm, table_hbm, out_hbm, idx_v, rows_v, sem):
    wid = lax.axis_index("s") * NUM_CORES + lax.axis_index("c")
    base = wid * PER_WORKER
    pltpu.sync_copy(ids_hbm.at[pl.ds(base, PER_WORKER)], idx_v)

    def body(j, carry):
        off = j * CHUNK
        pltpu.async_copy(
            table_hbm.at[idx_v.at[pl.ds(off, CHUNK)]], rows_v, sem
        ).wait()
        pltpu.sync_copy(rows_v, out_hbm.at[pl.ds(base + off, CHUNK)])
        return carry

    lax.fori_loop(0, NUM_CHUNKS, body, 0)


@jax.jit
def kernel(token_ids, embeddings):
    flat_ids = token_ids.reshape(-1).astype(jnp.int32)
    out = _gather_kernel(flat_ids, embeddings)
    return out.reshape(token_ids.shape + (EMBED_DIM,))


# speedup vs baseline: 1.1095x; 1.0065x over previous
import functools

import jax
import jax.numpy as jnp
from jax import lax
from jax.experimental import pallas as pl
from jax.experimental.pallas import tpu as pltpu
from jax.experimental.pallas import tpu_sc as plsc

NUM_TOKENS = 16384 * 50
EMBED_DIM = 32
NUM_CORES = 2
NUM_SUBCORES = 16
NUM_WORKERS = NUM_CORES * NUM_SUBCORES
PER_WORKER = NUM_TOKENS // NUM_WORKERS
CHUNK = 640
NUM_CHUNKS = PER_WORKER // CHUNK
K = 2
NUM_SG = NUM_CHUNKS // (2 * K)

_mesh = plsc.VectorSubcoreMesh(core_axis_name="c", subcore_axis_name="s")


@functools.partial(
    pl.kernel,
    mesh=_mesh,
    compiler_params=pltpu.CompilerParams(use_tc_tiling_on_sc=False),
    out_type=jax.ShapeDtypeStruct((NUM_TOKENS, EMBED_DIM), jnp.float32),
    scratch_types=[
        pltpu.VMEM((PER_WORKER,), jnp.int32),
        pltpu.VMEM((CHUNK, EMBED_DIM), jnp.float32),
        pltpu.VMEM((CHUNK, EMBED_DIM), jnp.float32),
        pltpu.VMEM((CHUNK, EMBED_DIM), jnp.float32),
        pltpu.VMEM((CHUNK, EMBED_DIM), jnp.float32),
        pltpu.SemaphoreType.DMA,
        pltpu.SemaphoreType.DMA,
        pltpu.SemaphoreType.DMA,
        pltpu.SemaphoreType.DMA,
    ],
)
def _gather_kernel(ids_hbm, table_hbm, out_hbm, idx_v, a0, a1, b0, b1,
                   sem_ga, sem_gb, sem_wa, sem_wb):
    wid = lax.axis_index("s") * NUM_CORES + lax.axis_index("c")
    base = wid * PER_WORKER
    pltpu.sync_copy(ids_hbm.at[pl.ds(base, PER_WORKER)], idx_v)

    bufs_a = [a0, a1]
    bufs_b = [b0, b1]

    def gather_start(chunk, buf, sem):
        pltpu.async_copy(table_hbm.at[idx_v.at[pl.ds(chunk * CHUNK, CHUNK)]],
                         buf, sem)

    def gather_wait(chunk, buf, sem):
        pltpu.make_async_copy(
            table_hbm.at[idx_v.at[pl.ds(chunk * CHUNK, CHUNK)]], buf, sem
        ).wait()

    def write_start(chunk, buf, sem):
        pltpu.async_copy(buf, out_hbm.at[pl.ds(base + chunk * CHUNK, CHUNK)],
                         sem)

    def write_wait(chunk, buf, sem):
        pltpu.make_async_copy(
            buf, out_hbm.at[pl.ds(base + chunk * CHUNK, CHUNK)], sem
        ).wait()

    for b in range(K):
        gather_start(b, bufs_a[b], sem_ga)

    def body(s, carry):
        c0 = s * 2 * K

        for b in range(K):
            gather_wait(c0 + b, bufs_a[b], sem_ga)

        @pl.when(s > 0)
        def _():
            for b in range(K):
                write_wait(c0 - K + b, bufs_b[b], sem_wb)

        for b in range(K):
            gather_start(c0 + K + b, bufs_b[b], sem_gb)
        for b in range(K):
            write_start(c0 + b, bufs_a[b], sem_wa)
        for b in range(K):
            gather_wait(c0 + K + b, bufs_b[b], sem_gb)
        for b in range(K):
            write_wait(c0 + b, bufs_a[b], sem_wa)

        @pl.when(s < NUM_SG - 1)
        def _():
            for b in range(K):
                gather_start(c0 + 2 * K + b, bufs_a[b], sem_ga)

        for b in range(K):
            write_start(c0 + K + b, bufs_b[b], sem_wb)
        return carry

    lax.fori_loop(0, NUM_SG, body, 0)

    for b in range(K):
        write_wait((NUM_SG - 1) * 2 * K + K + b, bufs_b[b], sem_wb)


@jax.jit
def kernel(token_ids, embeddings):
    flat_ids = token_ids.reshape(-1).astype(jnp.int32)
    out = _gather_kernel(flat_ids, embeddings)
    return out.reshape(token_ids.shape + (EMBED_DIM,))
